# Initial kernel scaffold; baseline (speedup 1.0000x reference)
import jax
import jax.numpy as jnp
from jax.experimental import pallas as pl


def kernel(numeric_features, team_ids, home_table, away_table, W1, b1, W2, b2, W3, b3):
    raise NotImplementedError("write your pallas kernel here")


# speedup vs baseline: 1.7500x; 1.7500x over previous
import functools

import jax
import jax.numpy as jnp
from jax import lax
from jax.experimental import pallas as pl
from jax.experimental.pallas import tpu as pltpu
from jax.experimental.pallas import tpu_sc as plsc

_NC, _NS = 2, 16
_NW = _NC * _NS
_B = 16384
_H1 = 64
_NUMERIC = 100
_EMBED = 128
_TEAMS = 1000
_BPW = 2 * _B // _NW


def _prep_body(ht_ref, at_ref, w1h_ref, w1a_ref, out_ref):
    out_ref[0] = jnp.dot(ht_ref[...], w1h_ref[...],
                         preferred_element_type=jnp.float32)
    out_ref[1] = jnp.dot(at_ref[...], w1a_ref[...],
                         preferred_element_type=jnp.float32)


_prep = pl.pallas_call(
    _prep_body,
    out_shape=jax.ShapeDtypeStruct((2, _TEAMS, _H1), jnp.float32),
)


_sc_mesh = plsc.VectorSubcoreMesh(core_axis_name="c", subcore_axis_name="s")


@functools.partial(
    pl.kernel,
    mesh=_sc_mesh,
    compiler_params=pltpu.CompilerParams(use_tc_tiling_on_sc=False),
    out_type=jax.ShapeDtypeStruct((2 * _B, _H1), jnp.float32),
    scratch_types=[
        pltpu.VMEM((_BPW,), jnp.int32),
        pltpu.VMEM((_BPW, _H1), jnp.float32),
        pltpu.SemaphoreType.DMA,
    ],
)
def _gather(table_hbm, idx_hbm, out_hbm, idx_v, rows_v, sem):
    wid = lax.axis_index("s") * _NC + lax.axis_index("c")
    base = wid * _BPW
    pltpu.sync_copy(idx_hbm.at[pl.ds(base, _BPW)], idx_v)
    pltpu.async_copy(table_hbm.at[idx_v], rows_v, sem).wait()
    pltpu.sync_copy(rows_v, out_hbm.at[pl.ds(base, _BPW)])


_BB = 2048


def _mlp_body(num_ref, g_ref, w1_ref, b1_ref, w2_ref, b2_ref, w3_ref, b3_ref,
              out_ref):
    x = jnp.dot(num_ref[...], w1_ref[...], preferred_element_type=jnp.float32)
    x = jnp.maximum(x + g_ref[0] + g_ref[1] + b1_ref[...], 0.0)
    x = jnp.maximum(
        jnp.dot(x, w2_ref[...], preferred_element_type=jnp.float32)
        + b2_ref[...], 0.0)
    out_ref[...] = (jnp.dot(x, w3_ref[...], preferred_element_type=jnp.float32)
                    + b3_ref[...])


_mlp = pl.pallas_call(
    _mlp_body,
    grid=(_B // _BB,),
    in_specs=[
        pl.BlockSpec((_BB, _NUMERIC), lambda i: (i, 0)),
        pl.BlockSpec((2, _BB, _H1), lambda i: (0, i, 0)),
        pl.BlockSpec((_NUMERIC, _H1), lambda i: (0, 0)),
        pl.BlockSpec((1, _H1), lambda i: (0, 0)),
        pl.BlockSpec((_H1, 32), lambda i: (0, 0)),
        pl.BlockSpec((1, 32), lambda i: (0, 0)),
        pl.BlockSpec((32, 2), lambda i: (0, 0)),
        pl.BlockSpec((1, 2), lambda i: (0, 0)),
    ],
    out_specs=pl.BlockSpec((_BB, 2), lambda i: (i, 0)),
    out_shape=jax.ShapeDtypeStruct((_B, 2), jnp.float32),
)


def kernel(numeric_features, team_ids, home_table, away_table,
           W1, b1, W2, b2, W3, b3):
    ids = team_ids.astype(jnp.int32)
    idx = jnp.concatenate([ids[:, 0], ids[:, 1] + _TEAMS])
    table = _prep(home_table, away_table,
                  W1[_NUMERIC:_NUMERIC + _EMBED],
                  W1[_NUMERIC + _EMBED:]).reshape(2 * _TEAMS, _H1)
    g = _gather(table, idx).reshape(2, _B, _H1)
    return _mlp(numeric_features, g, W1[:_NUMERIC], b1.reshape(1, _H1),
                W2, b2.reshape(1, 32), W3, b3.reshape(1, 2))


# speedup vs baseline: 2.2408x; 1.2804x over previous
import functools

import jax
import jax.numpy as jnp
from jax import lax
from jax.experimental import pallas as pl
from jax.experimental.pallas import tpu as pltpu
from jax.experimental.pallas import tpu_sc as plsc

_NC, _NS = 2, 16
_NW = _NC * _NS
_B = 16384
_H1 = 64
_NUMERIC = 100
_EMBED = 128
_TEAMS = 1000
_BPW = 2 * _B // _NW


def _prep_body(ht_ref, at_ref, w1h_ref, w1a_ref, out_ref):
    out_ref[0] = jnp.dot(ht_ref[...], w1h_ref[...],
                         preferred_element_type=jnp.float32)
    out_ref[1] = jnp.dot(at_ref[...], w1a_ref[...],
                         preferred_element_type=jnp.float32)


_prep = pl.pallas_call(
    _prep_body,
    out_shape=jax.ShapeDtypeStruct((2, _TEAMS, _H1), jnp.float32),
)


_sc_mesh = plsc.VectorSubcoreMesh(core_axis_name="c", subcore_axis_name="s")


@functools.partial(
    pl.kernel,
    mesh=_sc_mesh,
    compiler_params=pltpu.CompilerParams(use_tc_tiling_on_sc=False),
    out_type=jax.ShapeDtypeStruct((2 * _B, _H1), jnp.float32),
    scratch_types=[
        pltpu.VMEM((_BPW,), jnp.int32),
        pltpu.VMEM((_BPW, _H1), jnp.float32),
        pltpu.SemaphoreType.DMA,
    ],
)
def _gather(table_hbm, idx_hbm, out_hbm, idx_v, rows_v, sem):
    wid = lax.axis_index("s") * _NC + lax.axis_index("c")
    base = wid * _BPW
    pltpu.sync_copy(idx_hbm.at[pl.ds(base, _BPW)], idx_v)
    pltpu.async_copy(table_hbm.at[idx_v], rows_v, sem).wait()
    pltpu.sync_copy(rows_v, out_hbm.at[pl.ds(base, _BPW)])


_BB = 2048


def _mlp_body(numt_ref, g_ref, s_ref, w1_ref, b1_ref, w2t_ref, b2_ref,
              w3t_ref, b3_ref, out_ref):
    x = lax.dot_general(numt_ref[...], w1_ref[...],
                        (((0,), (0,)), ((), ())),
                        preferred_element_type=jnp.float32)
    gsum = jnp.dot(g_ref[...], s_ref[...], preferred_element_type=jnp.float32)
    x = jnp.maximum(x + gsum + b1_ref[...], 0.0)
    x = jnp.maximum(
        lax.dot_general(x, w2t_ref[...], (((1,), (1,)), ((), ())),
                        preferred_element_type=jnp.float32)
        + b2_ref[...], 0.0)
    out_ref[...] = (
        lax.dot_general(w3t_ref[...], x, (((1,), (1,)), ((), ())),
                        preferred_element_type=jnp.float32)
        + b3_ref[...])


_mlp = pl.pallas_call(
    _mlp_body,
    grid=(_B // _BB,),
    in_specs=[
        pl.BlockSpec((_NUMERIC, _BB), lambda i: (0, i)),
        pl.BlockSpec((_BB, 2 * _H1), lambda i: (i, 0)),
        pl.BlockSpec((2 * _H1, _H1), lambda i: (0, 0)),
        pl.BlockSpec((_NUMERIC, _H1), lambda i: (0, 0)),
        pl.BlockSpec((1, _H1), lambda i: (0, 0)),
        pl.BlockSpec((32, _H1), lambda i: (0, 0)),
        pl.BlockSpec((1, 32), lambda i: (0, 0)),
        pl.BlockSpec((2, 32), lambda i: (0, 0)),
        pl.BlockSpec((2, 1), lambda i: (0, 0)),
    ],
    out_specs=pl.BlockSpec((2, _BB), lambda i: (0, i)),
    out_shape=jax.ShapeDtypeStruct((2, _B), jnp.float32),
)


def kernel(numeric_features, team_ids, home_table, away_table,
           W1, b1, W2, b2, W3, b3):
    ids = team_ids.astype(jnp.int32)
    idx = (ids + jnp.array([0, _TEAMS], dtype=jnp.int32)).reshape(-1)
    table = _prep(home_table, away_table,
                  W1[_NUMERIC:_NUMERIC + _EMBED],
                  W1[_NUMERIC + _EMBED:]).reshape(2 * _TEAMS, _H1)
    g = _gather(table, idx).reshape(_B, 2 * _H1)
    eye = jnp.eye(_H1, dtype=jnp.float32)
    s = jnp.concatenate([eye, eye], axis=0)
    out_t = _mlp(numeric_features.T, g, s,
                 W1[:_NUMERIC], b1.reshape(1, _H1),
                 W2.T, b2.reshape(1, 32), W3.T, b3.reshape(2, 1))
    return out_t.T


# speedup vs baseline: 2.5859x; 1.1540x over previous
import functools

import jax
import jax.numpy as jnp
from jax import lax
from jax.experimental import pallas as pl
from jax.experimental.pallas import tpu as pltpu
from jax.experimental.pallas import tpu_sc as plsc

_NC, _NS = 2, 16
_NW = _NC * _NS
_B = 16384
_H1 = 64
_NUMERIC = 100
_EMBED = 128
_TEAMS = 1000
_SPW = _B // _NW


def _prep_body(ht_ref, at_ref, w1et_ref, out_ref):
    out_ref[0] = lax.dot_general(ht_ref[...], w1et_ref[:, :_EMBED],
                                 (((1,), (1,)), ((), ())),
                                 preferred_element_type=jnp.float32)
    out_ref[1] = lax.dot_general(at_ref[...], w1et_ref[:, _EMBED:],
                                 (((1,), (1,)), ((), ())),
                                 preferred_element_type=jnp.float32)


_prep = pl.pallas_call(
    _prep_body,
    out_shape=jax.ShapeDtypeStruct((2, _TEAMS, _H1), jnp.float32),
)


_sc_mesh = plsc.VectorSubcoreMesh(core_axis_name="c", subcore_axis_name="s")


@functools.partial(
    pl.kernel,
    mesh=_sc_mesh,
    compiler_params=pltpu.CompilerParams(use_tc_tiling_on_sc=False),
    out_type=jax.ShapeDtypeStruct((_B, 2 * _H1), jnp.float32),
    scratch_types=[
        pltpu.VMEM((_SPW,), jnp.int32),
        pltpu.VMEM((_SPW,), jnp.int32),
        pltpu.VMEM((_SPW, _H1), jnp.float32),
        pltpu.VMEM((_SPW, _H1), jnp.float32),
        pltpu.SemaphoreType.DMA,
    ],
)
def _gather(table_hbm, idx_hbm, out_hbm, idxh_v, idxa_v, rh_v, ra_v, sem):
    wid = lax.axis_index("s") * _NC + lax.axis_index("c")
    base = wid * _SPW
    pltpu.sync_copy(idx_hbm.at[pl.ds(base, _SPW)], idxh_v)
    pltpu.sync_copy(idx_hbm.at[pl.ds(_B + base, _SPW)], idxa_v)
    cp_h = pltpu.async_copy(table_hbm.at[idxh_v], rh_v, sem)
    cp_a = pltpu.async_copy(table_hbm.at[idxa_v], ra_v, sem)
    cp_h.wait()
    cp_a.wait()
    pltpu.sync_copy(rh_v, out_hbm.at[pl.ds(base, _SPW), pl.ds(0, _H1)])
    pltpu.sync_copy(ra_v, out_hbm.at[pl.ds(base, _SPW), pl.ds(_H1, _H1)])


_PBB = 4096


def _pproj_body(numt_ref, w1nt_ref, out_ref):
    out_ref[...] = lax.dot_general(numt_ref[...], w1nt_ref[...],
                                   (((0,), (1,)), ((), ())),
                                   preferred_element_type=jnp.float32)


_pproj = pl.pallas_call(
    _pproj_body,
    grid=(_B // _PBB,),
    in_specs=[
        pl.BlockSpec((_NUMERIC, _PBB), lambda i: (0, i)),
        pl.BlockSpec((_H1, _NUMERIC), lambda i: (0, 0)),
    ],
    out_specs=pl.BlockSpec((_PBB, _H1), lambda i: (i, 0)),
    out_shape=jax.ShapeDtypeStruct((_B, _H1), jnp.float32),
)


_BB = 2048


def _mlp_body(p_ref, g_ref, s_ref, b1_ref, w2t_ref, b2_ref, w3t_ref, b3_ref,
              out_ref):
    gsum = jnp.dot(g_ref[...], s_ref[...], preferred_element_type=jnp.float32)
    x = jnp.maximum(p_ref[...] + gsum + b1_ref[...], 0.0)
    x = jnp.maximum(
        lax.dot_general(x, w2t_ref[...], (((1,), (1,)), ((), ())),
                        preferred_element_type=jnp.float32)
        + b2_ref[...], 0.0)
    out_ref[...] = (
        lax.dot_general(w3t_ref[...], x, (((1,), (1,)), ((), ())),
                        preferred_element_type=jnp.float32)
        + b3_ref[...])


_mlp = pl.pallas_call(
    _mlp_body,
    grid=(_B // _BB,),
    in_specs=[
        pl.BlockSpec((_BB, _H1), lambda i: (i, 0)),
        pl.BlockSpec((_BB, 2 * _H1), lambda i: (i, 0)),
        pl.BlockSpec((2 * _H1, _H1), lambda i: (0, 0)),
        pl.BlockSpec((1, _H1), lambda i: (0, 0)),
        pl.BlockSpec((32, _H1), lambda i: (0, 0)),
        pl.BlockSpec((1, 32), lambda i: (0, 0)),
        pl.BlockSpec((2, 32), lambda i: (0, 0)),
        pl.BlockSpec((2, 1), lambda i: (0, 0)),
    ],
    out_specs=pl.BlockSpec((2, _BB), lambda i: (0, i)),
    out_shape=jax.ShapeDtypeStruct((2, _B), jnp.float32),
)


def kernel(numeric_features, team_ids, home_table, away_table,
           W1, b1, W2, b2, W3, b3):
    ids = team_ids.astype(jnp.int32)
    idx = jnp.concatenate([ids[:, 0], ids[:, 1] + _TEAMS])
    w1t = W1.T
    table = _prep(home_table, away_table,
                  w1t[:, _NUMERIC:]).reshape(2 * _TEAMS, _H1)
    g = _gather(table, idx)
    p = _pproj(numeric_features.T, w1t[:, :_NUMERIC])
    eye = jnp.eye(_H1, dtype=jnp.float32)
    s = jnp.concatenate([eye, eye], axis=0)
    out_t = _mlp(p, g, s, b1.reshape(1, _H1),
                 W2.T, b2.reshape(1, 32), W3.T, b3.reshape(2, 1))
    return out_t.T
